# baseline (speedup 1.0000x reference)
import functools

import jax
import jax.numpy as jnp
from jax import lax
from jax.experimental import pallas as pl
from jax.experimental.pallas import tpu as pltpu
from jax.experimental.pallas import tpu_sc as plsc

_NUM_PATCHES = 1024
_DIM = 768
_NC = 2
_NS = 16
_NW = _NC * _NS
_ROWS = _NUM_PATCHES // _NW


def _make_sc_bcast(batch):
    mesh = plsc.VectorSubcoreMesh(core_axis_name="c", subcore_axis_name="s")

    @functools.partial(
        pl.kernel,
        mesh=mesh,
        out_type=jax.ShapeDtypeStruct((batch, _NUM_PATCHES, _DIM), jnp.float32),
        scratch_types=[
            pltpu.VMEM((_ROWS, _DIM), jnp.float32),
            pltpu.SemaphoreType.DMA,
        ],
    )
    def sc_bcast(table_hbm, out_hbm, chunk_v, sem):
        wid = lax.axis_index("s") * _NC + lax.axis_index("c")
        base = wid * _ROWS
        pltpu.sync_copy(table_hbm.at[pl.ds(base, _ROWS)], chunk_v)

        def fire(b, carry):
            pltpu.make_async_copy(
                chunk_v, out_hbm.at[b, pl.ds(base, _ROWS)], sem
            ).start()
            return carry

        lax.fori_loop(0, batch, fire, 0)

        def drain(b, carry):
            pltpu.make_async_copy(
                chunk_v, out_hbm.at[0, pl.ds(base, _ROWS)], sem
            ).wait()
            return carry

        lax.fori_loop(0, batch, drain, 0)

    return sc_bcast


def kernel(x, table):
    return _make_sc_bcast(x.shape[0])(table)
